# baseline (speedup 1.0000x reference)
import functools

import jax
import jax.numpy as jnp
from jax import lax
from jax.experimental import pallas as pl
from jax.experimental.pallas import tpu as pltpu
from jax.experimental.pallas import tpu_sc as plsc

_N = 50000
_S = 5000
_S_PAD = 5120
_BS = 512
_NB = _S_PAD // _BS
_ES = 16384
_LB = 4096
_NLB = _ES // _LB
_B_PAD = 40960
_T_PRO = 0
_T_LINK0 = 1
_T_PAIR0 = _T_LINK0 + _NLB
_GRID = _T_PAIR0 + _NB * _NB
_HIGH = jax.lax.Precision.HIGHEST
_LOG2E = 1.4426950408889634


def _sc_gather_rows(tab, idx):
    n_rows, d = tab.shape
    b = idx.shape[0]
    info = plsc.get_sparse_core_info()
    nc, ns = info.num_cores, info.num_subcores
    nw = nc * ns
    bpw = b // nw
    chunk = 128
    nchunk = bpw // chunk
    mesh = plsc.VectorSubcoreMesh(core_axis_name="c", subcore_axis_name="s")

    sw = _S_PAD // bpw

    @functools.partial(
        pl.kernel,
        mesh=mesh,
        out_type=[
            jax.ShapeDtypeStruct((_S_PAD, d), jnp.float32),
            jax.ShapeDtypeStruct((b - _S_PAD, d), jnp.float32),
        ],
        compiler_params=pltpu.CompilerParams(use_tc_tiling_on_sc=False),
        scratch_types=[
            pltpu.VMEM((bpw,), jnp.int32),
            pltpu.VMEM((bpw, d), jnp.float32),
            pltpu.SemaphoreType.DMA,
        ],
    )
    def gather_kernel(tab_hbm, idx_hbm, outs_hbm, outl_hbm, idx_v, rows_v, sem):
        wid = lax.axis_index("s") * nc + lax.axis_index("c")
        base = wid * bpw
        pltpu.sync_copy(idx_hbm.at[pl.ds(base, bpw)], idx_v)
        copies = []
        for j in range(nchunk):
            copies.append(
                pltpu.async_copy(
                    tab_hbm.at[idx_v.at[pl.ds(j * chunk, chunk)]],
                    rows_v.at[pl.ds(j * chunk, chunk)],
                    sem,
                )
            )
        for c in copies:
            c.wait()

        @pl.when(wid < sw)
        def _ws():
            pltpu.sync_copy(rows_v, outs_hbm.at[pl.ds(base, bpw)])

        @pl.when(wid >= sw)
        def _wl():
            pltpu.sync_copy(rows_v, outl_hbm.at[pl.ds(base - _S_PAD, bpw)])

    return gather_kernel(tab, idx)


def _softmax_rows(x):
    m = jnp.max(x, axis=1, keepdims=True)
    e = jnp.exp(x - m)
    return e / jnp.sum(e, axis=1, keepdims=True)


def _tc_body(z1_ref, gate_ref, gl_ref, gs_ref, vc_ref, a_ref, out_ref,
             azc_ref,
             pxc_ref, pyc_ref, lbc_ref, pxr_ref, pyr_ref, lbr_ref, sm_ref):
    t = pl.program_id(0)

    @pl.when(t == _T_PRO)
    def _prologue():
        x = z1_ref[...]
        e = jnp.exp(x)
        li = lax.broadcasted_iota(jnp.int32, (128, 128), 0) // 8
        lj = lax.broadcasted_iota(jnp.int32, (128, 128), 1) // 8
        seg = (li == lj).astype(jnp.float32)
        s = lax.dot_general(e, seg, (((1,), (0,)), ((), ())),
                            precision=_HIGH)
        z = e / s
        gg = 1.0 / (1.0 + jnp.exp(-gate_ref[...]))
        zg = z * gg
        mbig = lax.dot_general(z, zg, (((0,), (0,)), ((), ())),
                               precision=_HIGH)
        csb = jnp.sum(zg, axis=0, keepdims=True)
        m = jnp.zeros((8, 8), jnp.float32)
        cs = jnp.zeros((1, 8), jnp.float32)
        for g in range(16):
            m = m + mbig[8 * g:8 * g + 8, 8 * g:8 * g + 8]
            cs = cs + csb[:, 8 * g:8 * g + 8]
        ms = m / cs
        azc = lax.dot_general(a_ref[...], ms, (((1,), (0,)), ((), ())),
                              precision=_HIGH)
        azc_ref[...] = azc

        grows = gs_ref[...]
        z = _softmax_rows(grows[:, 0:8])
        pts = lax.dot_general(z, azc, (((1,), (1,)), ((), ())),
                              precision=_HIGH)
        psc = pts * _LOG2E
        pxc = psc[:, 0:1]
        pyc = psc[:, 1:2]
        pxc_ref[...] = pxc
        pyc_ref[...] = pyc
        ridx = lax.broadcasted_iota(jnp.int32, (_S_PAD, 1), 0)
        valid = ridx < _S
        bcol = grows[:, 8:9]
        lb = jnp.where(valid, bcol * _LOG2E, -1e30)
        lbc_ref[...] = lb
        ebv = jnp.where(valid, jnp.exp(bcol), 0.0)
        pxr_ref[...] = jnp.transpose(pxc, (1, 0))
        pyr_ref[...] = jnp.transpose(pyc, (1, 0))
        lbr_ref[...] = jnp.transpose(lb, (1, 0))
        sm_ref[1] = jnp.sum(ebv * ebv)
        sm_ref[0] = 0.0
        sm_ref[2] = 0.0

    @pl.when((t >= _T_LINK0) & (t < _T_PAIR0))
    def _link():
        gl = gl_ref[...]
        azc = azc_ref[...]
        a0 = azc[0:1, :]
        a1 = azc[1:2, :]
        zi = _softmax_rows(gl[:, 0:8])
        zj = _softmax_rows(gl[:, 16:24])
        dx = (jnp.sum(zi * a0, axis=1, keepdims=True)
              - jnp.sum(zj * a0, axis=1, keepdims=True) + 1e-6)
        dy = (jnp.sum(zi * a1, axis=1, keepdims=True)
              - jnp.sum(zj * a1, axis=1, keepdims=True) + 1e-6)
        s2 = jnp.maximum(dx * dx + dy * dy, 1e-30)
        dl = s2 * lax.rsqrt(s2)
        sterm = gl[:, 8:9] + gl[:, 24:25] - dl
        sm_ref[2] += jnp.sum(sterm * vc_ref[...])

    @pl.when(t >= _T_PAIR0)
    def _pair_blocks():
        q = t - _T_PAIR0
        bi = q // _NB
        bj = q % _NB

        @pl.when(bj >= bi)
        def _block():
            xi = pxc_ref[pl.ds(bi * _BS, _BS), :]
            yi = pyc_ref[pl.ds(bi * _BS, _BS), :]
            lbi = lbc_ref[pl.ds(bi * _BS, _BS), :]
            xj = pxr_ref[:, pl.ds(bj * _BS, _BS)]
            yj = pyr_ref[:, pl.ds(bj * _BS, _BS)]
            lbj = lbr_ref[:, pl.ds(bj * _BS, _BS)]
            dx = xi - xj
            dy = yi - yj
            eps = jnp.float32(1e-12 * _LOG2E * _LOG2E)
            d2 = dx * dx + dy * dy + eps
            d = d2 * lax.rsqrt(d2)
            w = jnp.exp2((lbi + lbj) - d)
            part = jnp.sum(w)
            sm_ref[0] += jnp.where(bj > bi, 2.0, 1.0) * part

    @pl.when(t == _GRID - 1)
    def _epilogue():
        diag = jnp.exp(-jnp.sqrt(jnp.float32(1e-12)))
        z_pdist1 = 0.5 * (sm_ref[0] - diag * sm_ref[1])
        out_ref[...] = jnp.full((1, 1), sm_ref[2] - z_pdist1, jnp.float32)


def _tc_main(z1p, gatep, glink, gs, vc, a):
    def _imap(t):
        return (jnp.clip(t - _T_LINK0, 0, _NLB - 1), 0)

    return pl.pallas_call(
        _tc_body,
        grid=(_GRID,),
        in_specs=[
            pl.BlockSpec((3125, 128), lambda t: (0, 0)),
            pl.BlockSpec((3125, 128), lambda t: (0, 0)),
            pl.BlockSpec((_LB, 32), _imap),
            pl.BlockSpec((_S_PAD, 16), lambda t: (0, 0)),
            pl.BlockSpec((_LB, 1), _imap),
            pl.BlockSpec((2, 8), lambda t: (0, 0)),
        ],
        out_specs=pl.BlockSpec((1, 1), lambda t: (0, 0)),
        out_shape=jax.ShapeDtypeStruct((1, 1), jnp.float32),
        scratch_shapes=[
            pltpu.VMEM((2, 8), jnp.float32),
            pltpu.VMEM((_S_PAD, 1), jnp.float32),
            pltpu.VMEM((_S_PAD, 1), jnp.float32),
            pltpu.VMEM((_S_PAD, 1), jnp.float32),
            pltpu.VMEM((1, _S_PAD), jnp.float32),
            pltpu.VMEM((1, _S_PAD), jnp.float32),
            pltpu.VMEM((1, _S_PAD), jnp.float32),
            pltpu.SMEM((3,), jnp.float32),
        ],
        compiler_params=pltpu.CompilerParams(
            dimension_semantics=("arbitrary",),
        ),
    )(z1p, gatep, glink, gs, vc, a)


def kernel(latent_z1, beta, A, Gate, valueC, sample_idx,
           sparse_sample_i, sparse_sample_j):
    n = latent_z1.shape[0]
    s = sample_idx.shape[0]
    es = sparse_sample_i.shape[0]
    si = sample_idx.astype(jnp.int32)
    ii = sparse_sample_i.astype(jnp.int32)
    jj = sparse_sample_j.astype(jnp.int32)
    pairs = jnp.stack([ii, jj], axis=1).reshape(-1)
    idx = jnp.concatenate([
        si,
        jnp.zeros((_S_PAD - s,), jnp.int32),
        pairs,
        jnp.zeros((_B_PAD - _S_PAD - 2 * es,), jnp.int32),
    ])
    tab = jnp.concatenate(
        [latent_z1, beta[:, None], jnp.zeros((n, 7), jnp.float32)], axis=1)
    gs, gl = _sc_gather_rows(tab, idx)
    glink = gl.reshape((_B_PAD - _S_PAD) // 2, 32)
    z1p = latent_z1.reshape(n // 16, 128)
    gatep = Gate.reshape(n // 16, 128)
    out = _tc_main(z1p, gatep, glink, gs, valueC[:, None], A)
    return out[0, 0]


# speedup vs baseline: 1.0670x; 1.0670x over previous
import functools

import jax
import jax.numpy as jnp
from jax import lax
from jax.experimental import pallas as pl
from jax.experimental.pallas import tpu as pltpu
from jax.experimental.pallas import tpu_sc as plsc

_N = 50000
_S = 5000
_S_PAD = 5120
_BS = 512
_NB = _S_PAD // _BS
_ES = 16384
_LB = 4096
_NLB = _ES // _LB
_B_PAD = 40960
_T_PRO = 0
_T_LINK0 = 1
_T_PAIR0 = _T_LINK0 + _NLB
_GRID = _T_PAIR0 + _NB * _NB
_HIGH = jax.lax.Precision.HIGHEST
_LOG2E = 1.4426950408889634


def _sc_gather_rows(tab, idx):
    n_rows, d = tab.shape
    b = idx.shape[0]
    info = plsc.get_sparse_core_info()
    nc, ns = info.num_cores, info.num_subcores
    nw = nc * ns
    bpw = b // nw
    chunk = 128
    nchunk = bpw // chunk
    mesh = plsc.VectorSubcoreMesh(core_axis_name="c", subcore_axis_name="s")

    @functools.partial(
        pl.kernel,
        mesh=mesh,
        out_type=jax.ShapeDtypeStruct((b, d), jnp.float32),
        compiler_params=pltpu.CompilerParams(use_tc_tiling_on_sc=False),
        scratch_types=[
            pltpu.VMEM((bpw,), jnp.int32),
            pltpu.VMEM((bpw, d), jnp.float32),
            pltpu.SemaphoreType.DMA,
        ],
    )
    def gather_kernel(tab_hbm, idx_hbm, out_hbm, idx_v, rows_v, sem):
        wid = lax.axis_index("s") * nc + lax.axis_index("c")
        base = wid * bpw
        pltpu.sync_copy(idx_hbm.at[pl.ds(base, bpw)], idx_v)
        copies = []
        for j in range(nchunk):
            copies.append(
                pltpu.async_copy(
                    tab_hbm.at[idx_v.at[pl.ds(j * chunk, chunk)]],
                    rows_v.at[pl.ds(j * chunk, chunk)],
                    sem,
                )
            )
        for c in copies:
            c.wait()
        pltpu.sync_copy(rows_v, out_hbm.at[pl.ds(base, bpw)])

    return gather_kernel(tab, idx)


def _softmax_rows(x):
    m = jnp.max(x, axis=1, keepdims=True)
    e = jnp.exp(x - m)
    return e / jnp.sum(e, axis=1, keepdims=True)


def _tc_body(z1_ref, gate_ref, gi_ref, gj_ref, gs_ref, vc_ref, a_ref, out_ref,
             azc_ref,
             pxc_ref, pyc_ref, lbc_ref, pxr_ref, pyr_ref, lbr_ref, sm_ref):
    t = pl.program_id(0)

    @pl.when(t == _T_PRO)
    def _prologue():
        x = z1_ref[...]
        e = jnp.exp(x)
        li = lax.broadcasted_iota(jnp.int32, (128, 128), 0) // 8
        lj = lax.broadcasted_iota(jnp.int32, (128, 128), 1) // 8
        seg = (li == lj).astype(jnp.float32)
        s = lax.dot_general(e, seg, (((1,), (0,)), ((), ())),
                            precision=_HIGH)
        z = e / s
        gg = 1.0 / (1.0 + jnp.exp(-gate_ref[...]))
        zg = z * gg
        mbig = lax.dot_general(z, zg, (((0,), (0,)), ((), ())),
                               precision=_HIGH)
        csb = jnp.sum(zg, axis=0, keepdims=True)
        m = jnp.zeros((8, 8), jnp.float32)
        cs = jnp.zeros((1, 8), jnp.float32)
        for g in range(16):
            m = m + mbig[8 * g:8 * g + 8, 8 * g:8 * g + 8]
            cs = cs + csb[:, 8 * g:8 * g + 8]
        ms = m / cs
        azc = lax.dot_general(a_ref[...], ms, (((1,), (0,)), ((), ())),
                              precision=_HIGH)
        azc_ref[...] = azc

        grows = gs_ref[...][0:_S_PAD]
        z = _softmax_rows(grows[:, 0:8])
        pts = lax.dot_general(z, azc, (((1,), (1,)), ((), ())),
                              precision=_HIGH)
        psc = pts * _LOG2E
        pxc = psc[:, 0:1]
        pyc = psc[:, 1:2]
        pxc_ref[...] = pxc
        pyc_ref[...] = pyc
        ridx = lax.broadcasted_iota(jnp.int32, (_S_PAD, 1), 0)
        valid = ridx < _S
        bcol = grows[:, 8:9]
        lb = jnp.where(valid, bcol * _LOG2E, -1e30)
        lbc_ref[...] = lb
        ebv = jnp.where(valid, jnp.exp(bcol), 0.0)
        pxr_ref[...] = jnp.transpose(pxc, (1, 0))
        pyr_ref[...] = jnp.transpose(pyc, (1, 0))
        lbr_ref[...] = jnp.transpose(lb, (1, 0))
        sm_ref[1] = jnp.sum(ebv * ebv)
        sm_ref[0] = 0.0
        sm_ref[2] = 0.0

    @pl.when((t >= _T_LINK0) & (t < _T_PAIR0))
    def _link():
        gi = gi_ref[...]
        gj = gj_ref[...]
        azc = azc_ref[...]
        a0 = azc[0:1, :]
        a1 = azc[1:2, :]
        zi = _softmax_rows(gi[:, 0:8])
        zj = _softmax_rows(gj[:, 0:8])
        dx = (jnp.sum(zi * a0, axis=1, keepdims=True)
              - jnp.sum(zj * a0, axis=1, keepdims=True) + 1e-6)
        dy = (jnp.sum(zi * a1, axis=1, keepdims=True)
              - jnp.sum(zj * a1, axis=1, keepdims=True) + 1e-6)
        s2 = jnp.maximum(dx * dx + dy * dy, 1e-30)
        dl = s2 * lax.rsqrt(s2)
        sterm = gi[:, 8:9] + gj[:, 8:9] - dl
        sm_ref[2] += jnp.sum(sterm * vc_ref[...])

    @pl.when(t >= _T_PAIR0)
    def _pair_blocks():
        q = t - _T_PAIR0
        bi = q // _NB
        bj = q % _NB

        @pl.when(bj >= bi)
        def _block():
            xi = pxc_ref[pl.ds(bi * _BS, _BS), :]
            yi = pyc_ref[pl.ds(bi * _BS, _BS), :]
            lbi = lbc_ref[pl.ds(bi * _BS, _BS), :]
            xj = pxr_ref[:, pl.ds(bj * _BS, _BS)]
            yj = pyr_ref[:, pl.ds(bj * _BS, _BS)]
            lbj = lbr_ref[:, pl.ds(bj * _BS, _BS)]
            dx = xi - xj
            dy = yi - yj
            eps = jnp.float32(1e-12 * _LOG2E * _LOG2E)
            d2 = dx * dx + dy * dy + eps
            d = d2 * lax.rsqrt(d2)
            w = jnp.exp2((lbi + lbj) - d)
            part = jnp.sum(w)
            sm_ref[0] += jnp.where(bj > bi, 2.0, 1.0) * part

    @pl.when(t == _GRID - 1)
    def _epilogue():
        diag = jnp.exp(-jnp.sqrt(jnp.float32(1e-12)))
        z_pdist1 = 0.5 * (sm_ref[0] - diag * sm_ref[1])
        out_ref[...] = jnp.full((1, 1), sm_ref[2] - z_pdist1, jnp.float32)


def _tc_main(z1p, gatep, grows, vc, a):
    def _imap(t):
        return (jnp.clip(t - _T_LINK0, 0, _NLB - 1), 0)

    def _jmap(t):
        return (_NLB + jnp.clip(t - _T_LINK0, 0, _NLB - 1), 0)

    return pl.pallas_call(
        _tc_body,
        grid=(_GRID,),
        in_specs=[
            pl.BlockSpec((3125, 128), lambda t: (0, 0)),
            pl.BlockSpec((3125, 128), lambda t: (0, 0)),
            pl.BlockSpec((_LB, 16), _imap),
            pl.BlockSpec((_LB, 16), _jmap),
            pl.BlockSpec((8192, 16), lambda t: (4, 0)),
            pl.BlockSpec((_LB, 1), _imap),
            pl.BlockSpec((2, 8), lambda t: (0, 0)),
        ],
        out_specs=pl.BlockSpec((1, 1), lambda t: (0, 0)),
        out_shape=jax.ShapeDtypeStruct((1, 1), jnp.float32),
        scratch_shapes=[
            pltpu.VMEM((2, 8), jnp.float32),
            pltpu.VMEM((_S_PAD, 1), jnp.float32),
            pltpu.VMEM((_S_PAD, 1), jnp.float32),
            pltpu.VMEM((_S_PAD, 1), jnp.float32),
            pltpu.VMEM((1, _S_PAD), jnp.float32),
            pltpu.VMEM((1, _S_PAD), jnp.float32),
            pltpu.VMEM((1, _S_PAD), jnp.float32),
            pltpu.SMEM((3,), jnp.float32),
        ],
        compiler_params=pltpu.CompilerParams(
            dimension_semantics=("arbitrary",),
        ),
    )(z1p, gatep, grows, grows, grows, vc, a)


def kernel(latent_z1, beta, A, Gate, valueC, sample_idx,
           sparse_sample_i, sparse_sample_j):
    n = latent_z1.shape[0]
    s = sample_idx.shape[0]
    es = sparse_sample_i.shape[0]
    si = sample_idx.astype(jnp.int32)
    ii = sparse_sample_i.astype(jnp.int32)
    jj = sparse_sample_j.astype(jnp.int32)
    idx = jnp.concatenate([
        ii,
        jj,
        si,
        jnp.zeros((_B_PAD - 2 * es - s,), jnp.int32),
    ])
    tab = jnp.concatenate(
        [latent_z1, beta[:, None], jnp.zeros((n, 7), jnp.float32)], axis=1)
    grows = _sc_gather_rows(tab, idx)
    z1p = latent_z1.reshape(n // 16, 128)
    gatep = Gate.reshape(n // 16, 128)
    out = _tc_main(z1p, gatep, grows, valueC[:, None], A)
    return out[0, 0]


# speedup vs baseline: 1.0839x; 1.0158x over previous
import functools

import jax
import jax.numpy as jnp
from jax import lax
from jax.experimental import pallas as pl
from jax.experimental.pallas import tpu as pltpu
from jax.experimental.pallas import tpu_sc as plsc

_N = 50000
_S = 5000
_S_PAD = 5120
_BS = 512
_NB = _S_PAD // _BS
_ES = 16384
_LB = 4096
_NLB = _ES // _LB
_B_PAD = 40960
_T_PRO = 0
_T_LINK0 = 1
_T_PAIR0 = _T_LINK0 + _NLB
_NTRI = _NB * (_NB + 1) // 2
_GRID = _T_PAIR0 + _NTRI
_TRI_OFF = [r * _NB - r * (r - 1) // 2 for r in range(_NB)]
_HIGH = jax.lax.Precision.HIGHEST
_LOG2E = 1.4426950408889634


def _sc_gather_rows(tab, idx):
    n_rows, d = tab.shape
    b = idx.shape[0]
    info = plsc.get_sparse_core_info()
    nc, ns = info.num_cores, info.num_subcores
    nw = nc * ns
    bpw = b // nw
    chunk = 128
    nchunk = bpw // chunk
    mesh = plsc.VectorSubcoreMesh(core_axis_name="c", subcore_axis_name="s")

    @functools.partial(
        pl.kernel,
        mesh=mesh,
        out_type=jax.ShapeDtypeStruct((b, d), jnp.float32),
        compiler_params=pltpu.CompilerParams(use_tc_tiling_on_sc=False),
        scratch_types=[
            pltpu.VMEM((bpw,), jnp.int32),
            pltpu.VMEM((bpw, d), jnp.float32),
            pltpu.SemaphoreType.DMA,
        ],
    )
    def gather_kernel(tab_hbm, idx_hbm, out_hbm, idx_v, rows_v, sem):
        wid = lax.axis_index("s") * nc + lax.axis_index("c")
        base = wid * bpw
        pltpu.sync_copy(idx_hbm.at[pl.ds(base, bpw)], idx_v)
        copies = []
        for j in range(nchunk):
            copies.append(
                pltpu.async_copy(
                    tab_hbm.at[idx_v.at[pl.ds(j * chunk, chunk)]],
                    rows_v.at[pl.ds(j * chunk, chunk)],
                    sem,
                )
            )
        for c in copies:
            c.wait()
        pltpu.sync_copy(rows_v, out_hbm.at[pl.ds(base, bpw)])

    return gather_kernel(tab, idx)


def _softmax_rows(x):
    m = jnp.max(x, axis=1, keepdims=True)
    e = jnp.exp(x - m)
    return e / jnp.sum(e, axis=1, keepdims=True)


def _tc_body(z1_ref, gate_ref, gi_ref, gj_ref, gs_ref, vc_ref, a_ref, out_ref,
             azc_ref,
             pxc_ref, pyc_ref, lbc_ref, pxr_ref, pyr_ref, lbr_ref, sm_ref):
    t = pl.program_id(0)

    @pl.when(t == _T_PRO)
    def _prologue():
        x = z1_ref[...]
        e = jnp.exp(x)
        li = lax.broadcasted_iota(jnp.int32, (128, 128), 0) // 8
        lj = lax.broadcasted_iota(jnp.int32, (128, 128), 1) // 8
        seg = (li == lj).astype(jnp.float32)
        s = lax.dot_general(e, seg, (((1,), (0,)), ((), ())),
                            precision=_HIGH)
        z = e / s
        gg = 1.0 / (1.0 + jnp.exp(-gate_ref[...]))
        zg = z * gg
        mbig = lax.dot_general(z, zg, (((0,), (0,)), ((), ())),
                               precision=_HIGH)
        csb = jnp.sum(zg, axis=0, keepdims=True)
        m = jnp.zeros((8, 8), jnp.float32)
        cs = jnp.zeros((1, 8), jnp.float32)
        for g in range(16):
            m = m + mbig[8 * g:8 * g + 8, 8 * g:8 * g + 8]
            cs = cs + csb[:, 8 * g:8 * g + 8]
        ms = m / cs
        azc = lax.dot_general(a_ref[...], ms, (((1,), (0,)), ((), ())),
                              precision=_HIGH)
        azc_ref[...] = azc

        grows = gs_ref[...][0:_S_PAD]
        z = _softmax_rows(grows[:, 0:8])
        pts = lax.dot_general(z, azc, (((1,), (1,)), ((), ())),
                              precision=_HIGH)
        psc = pts * _LOG2E
        pxc = psc[:, 0:1]
        pyc = psc[:, 1:2]
        pxc_ref[...] = pxc
        pyc_ref[...] = pyc
        ridx = lax.broadcasted_iota(jnp.int32, (_S_PAD, 1), 0)
        valid = ridx < _S
        bcol = grows[:, 8:9]
        lb = jnp.where(valid, bcol * _LOG2E, -1e30)
        lbc_ref[...] = lb
        ebv = jnp.where(valid, jnp.exp(bcol), 0.0)
        pxr_ref[...] = jnp.transpose(pxc, (1, 0))
        pyr_ref[...] = jnp.transpose(pyc, (1, 0))
        lbr_ref[...] = jnp.transpose(lb, (1, 0))
        sm_ref[1] = jnp.sum(ebv * ebv)
        sm_ref[0] = 0.0
        sm_ref[2] = 0.0

    @pl.when((t >= _T_LINK0) & (t < _T_PAIR0))
    def _link():
        gi = gi_ref[...]
        gj = gj_ref[...]
        azc = azc_ref[...]
        a0 = azc[0:1, :]
        a1 = azc[1:2, :]
        zi = _softmax_rows(gi[:, 0:8])
        zj = _softmax_rows(gj[:, 0:8])
        dx = (jnp.sum(zi * a0, axis=1, keepdims=True)
              - jnp.sum(zj * a0, axis=1, keepdims=True) + 1e-6)
        dy = (jnp.sum(zi * a1, axis=1, keepdims=True)
              - jnp.sum(zj * a1, axis=1, keepdims=True) + 1e-6)
        s2 = jnp.maximum(dx * dx + dy * dy, 1e-30)
        dl = s2 * lax.rsqrt(s2)
        sterm = gi[:, 8:9] + gj[:, 8:9] - dl
        sm_ref[2] += jnp.sum(sterm * vc_ref[...])

    @pl.when(t >= _T_PAIR0)
    def _pair_blocks():
        q = t - _T_PAIR0
        bi = jnp.int32(0)
        for r in range(1, _NB):
            bi = bi + (q >= _TRI_OFF[r]).astype(jnp.int32)
        off = jnp.int32(0)
        for r in range(1, _NB):
            off = jnp.where(bi == r, jnp.int32(_TRI_OFF[r]), off)
        bj = bi + (q - off)
        xi = pxc_ref[pl.ds(bi * _BS, _BS), :]
        yi = pyc_ref[pl.ds(bi * _BS, _BS), :]
        lbi = lbc_ref[pl.ds(bi * _BS, _BS), :]
        xj = pxr_ref[:, pl.ds(bj * _BS, _BS)]
        yj = pyr_ref[:, pl.ds(bj * _BS, _BS)]
        lbj = lbr_ref[:, pl.ds(bj * _BS, _BS)]
        dx = xi - xj
        dy = yi - yj
        eps = jnp.float32(1e-12 * _LOG2E * _LOG2E)
        d2 = dx * dx + dy * dy + eps
        d = d2 * lax.rsqrt(d2)
        w = jnp.exp2((lbi + lbj) - d)
        part = jnp.sum(w)
        sm_ref[0] += jnp.where(bj > bi, 2.0, 1.0) * part

    @pl.when(t == _GRID - 1)
    def _epilogue():
        diag = jnp.exp(-jnp.sqrt(jnp.float32(1e-12)))
        z_pdist1 = 0.5 * (sm_ref[0] - diag * sm_ref[1])
        out_ref[...] = jnp.full((1, 1), sm_ref[2] - z_pdist1, jnp.float32)


def _tc_main(z1p, gatep, grows, vc, a):
    def _imap(t):
        return (jnp.clip(t - _T_LINK0, 0, _NLB - 1), 0)

    def _jmap(t):
        return (_NLB + jnp.clip(t - _T_LINK0, 0, _NLB - 1), 0)

    return pl.pallas_call(
        _tc_body,
        grid=(_GRID,),
        in_specs=[
            pl.BlockSpec((3125, 128), lambda t: (0, 0)),
            pl.BlockSpec((3125, 128), lambda t: (0, 0)),
            pl.BlockSpec((_LB, 16), _imap),
            pl.BlockSpec((_LB, 16), _jmap),
            pl.BlockSpec((8192, 16), lambda t: (4, 0)),
            pl.BlockSpec((_LB, 1), _imap),
            pl.BlockSpec((2, 8), lambda t: (0, 0)),
        ],
        out_specs=pl.BlockSpec((1, 1), lambda t: (0, 0)),
        out_shape=jax.ShapeDtypeStruct((1, 1), jnp.float32),
        scratch_shapes=[
            pltpu.VMEM((2, 8), jnp.float32),
            pltpu.VMEM((_S_PAD, 1), jnp.float32),
            pltpu.VMEM((_S_PAD, 1), jnp.float32),
            pltpu.VMEM((_S_PAD, 1), jnp.float32),
            pltpu.VMEM((1, _S_PAD), jnp.float32),
            pltpu.VMEM((1, _S_PAD), jnp.float32),
            pltpu.VMEM((1, _S_PAD), jnp.float32),
            pltpu.SMEM((3,), jnp.float32),
        ],
        compiler_params=pltpu.CompilerParams(
            dimension_semantics=("arbitrary",),
        ),
    )(z1p, gatep, grows, grows, grows, vc, a)


def kernel(latent_z1, beta, A, Gate, valueC, sample_idx,
           sparse_sample_i, sparse_sample_j):
    n = latent_z1.shape[0]
    s = sample_idx.shape[0]
    es = sparse_sample_i.shape[0]
    si = sample_idx.astype(jnp.int32)
    ii = sparse_sample_i.astype(jnp.int32)
    jj = sparse_sample_j.astype(jnp.int32)
    idx = jnp.concatenate([
        ii,
        jj,
        si,
        jnp.zeros((_B_PAD - 2 * es - s,), jnp.int32),
    ])
    tab = jnp.concatenate(
        [latent_z1, beta[:, None], jnp.zeros((n, 7), jnp.float32)], axis=1)
    grows = _sc_gather_rows(tab, idx)
    z1p = latent_z1.reshape(n // 16, 128)
    gatep = Gate.reshape(n // 16, 128)
    out = _tc_main(z1p, gatep, grows, valueC[:, None], A)
    return out[0, 0]
